# baseline (speedup 1.0000x reference)
import functools

import jax
import jax.numpy as jnp
from jax import lax
from jax.experimental import pallas as pl
from jax.experimental.pallas import tpu as pltpu
from jax.experimental.pallas import tpu_sc as plsc

NUM_CORES = 2
NUM_SUBCORES = 16
NW = NUM_CORES * NUM_SUBCORES
B = 16384
D = 128
RPW = B // NW
CHUNK = 128
CPW = RPW // CHUNK
NBUF = 7
NCH = 3 * CPW


def _transe_body(h_idx, r_idx, t_idx, ent_hbm, rel_hbm,
                 h_out, r_out, t_out,
                 idx_v, bufs, isem, gsem, wsem):
    wid = lax.axis_index("s") * NUM_CORES + lax.axis_index("c")
    base = wid * RPW

    row0 = wid * CPW
    iw = [pltpu.async_copy(idx_hbm.at[pl.ds(row0, CPW)], idx_v.at[tbl_i], isem)
          for tbl_i, idx_hbm in enumerate((h_idx, r_idx, t_idx))]
    for c in iw:
        c.wait()

    tables = (ent_hbm, rel_hbm, ent_hbm)
    outs = (h_out, r_out, t_out)

    def start_gather(c):
        tbl_i, j = divmod(c, CPW)
        return pltpu.async_copy(
            tables[tbl_i].at[idx_v.at[tbl_i].at[j]],
            bufs.at[c % NBUF], gsem)

    def start_wb(c):
        tbl_i, j = divmod(c, CPW)
        return pltpu.async_copy(
            bufs.at[c % NBUF], outs[tbl_i].at[pl.ds(base + j * CHUNK, CHUNK)], wsem)

    g = [None] * NCH
    w = [None] * NCH
    for c in range(NBUF):
        g[c] = start_gather(c)
    for c in range(NCH):
        if 0 < c <= NCH - NBUF:
            w[c - 1].wait()
            g[c - 1 + NBUF] = start_gather(c - 1 + NBUF)
        g[c].wait()
        w[c] = start_wb(c)
    for c in range(max(0, NCH - NBUF), NCH):
        w[c].wait()


def _transe(h, r, t, entity_emb, relation_emb):
    mesh = plsc.VectorSubcoreMesh(core_axis_name="c", subcore_axis_name="s")
    out_t = (jax.ShapeDtypeStruct((B, D), jnp.float32),) * 3
    run = functools.partial(
        pl.kernel, mesh=mesh,
        out_type=out_t,
        scratch_types=[
            pltpu.VMEM((3, CPW, CHUNK), jnp.int32),
            pltpu.VMEM((NBUF, CHUNK, D), jnp.float32),
            pltpu.SemaphoreType.DMA,
            pltpu.SemaphoreType.DMA,
            pltpu.SemaphoreType.DMA,
        ],
    )(_transe_body)
    return run(h, r, t, entity_emb, relation_emb)


def kernel(h, r, t, entity_emb, relation_emb):
    h2 = h.astype(jnp.int32).reshape(B // CHUNK, CHUNK)
    r2 = r.astype(jnp.int32).reshape(B // CHUNK, CHUNK)
    t2 = t.astype(jnp.int32).reshape(B // CHUNK, CHUNK)
    return _transe(h2, r2, t2, entity_emb, relation_emb)


# speedup vs baseline: 1.0436x; 1.0436x over previous
import functools

import jax
import jax.numpy as jnp
from jax import lax
from jax.experimental import pallas as pl
from jax.experimental.pallas import tpu as pltpu
from jax.experimental.pallas import tpu_sc as plsc

NUM_CORES = 2
NUM_SUBCORES = 16
NW = NUM_CORES * NUM_SUBCORES
B = 16384
D = 128
RPW = B // NW
CHUNK = 128
CPW = RPW // CHUNK
PAIR = 2 * CHUNK
PPT = CPW // 2
NPAIR = 3 * PPT
NBUF = 3


def _transe_body(h_idx, r_idx, t_idx, ent_hbm, rel_hbm,
                 h_out, r_out, t_out,
                 idx_v, bufs, gsem, wsem):
    wid = lax.axis_index("s") * NUM_CORES + lax.axis_index("c")
    base = wid * RPW
    row0 = wid * CPW

    for tbl_i, idx_hbm in enumerate((h_idx, r_idx, t_idx)):
        pltpu.sync_copy(idx_hbm.at[pl.ds(row0, CPW)], idx_v.at[tbl_i])

    tables = (ent_hbm, rel_hbm, ent_hbm)
    outs = (h_out, r_out, t_out)

    def start_pair(p):
        tbl_i, half = divmod(p, PPT)
        return [pltpu.async_copy(
                    tables[tbl_i].at[idx_v.at[tbl_i].at[half * 2 + k]],
                    bufs.at[p % NBUF].at[pl.ds(k * CHUNK, CHUNK)], gsem)
                for k in range(2)]

    def start_wb(p):
        tbl_i, half = divmod(p, PPT)
        return pltpu.async_copy(
            bufs.at[p % NBUF], outs[tbl_i].at[pl.ds(base + half * PAIR, PAIR)], wsem)

    g = [None] * NPAIR
    w = [None] * NPAIR
    for p in range(NBUF):
        g[p] = start_pair(p)
    for p in range(NPAIR):
        if 0 < p <= NPAIR - NBUF:
            w[p - 1].wait()
            g[p - 1 + NBUF] = start_pair(p - 1 + NBUF)
        for hnd in g[p]:
            hnd.wait()
        w[p] = start_wb(p)
    for p in range(max(0, NPAIR - NBUF), NPAIR):
        w[p].wait()


def _transe(h2, r2, t2, entity_emb, relation_emb):
    mesh = plsc.VectorSubcoreMesh(core_axis_name="c", subcore_axis_name="s")
    out_t = (jax.ShapeDtypeStruct((B, D), jnp.float32),) * 3
    run = functools.partial(
        pl.kernel, mesh=mesh,
        out_type=out_t,
        scratch_types=[
            pltpu.VMEM((3, CPW, CHUNK), jnp.int32),
            pltpu.VMEM((NBUF, PAIR, D), jnp.float32),
            pltpu.SemaphoreType.DMA,
            pltpu.SemaphoreType.DMA,
        ],
    )(_transe_body)
    return run(h2, r2, t2, entity_emb, relation_emb)


def kernel(h, r, t, entity_emb, relation_emb):
    h2 = h.astype(jnp.int32).reshape(B // CHUNK, CHUNK)
    r2 = r.astype(jnp.int32).reshape(B // CHUNK, CHUNK)
    t2 = t.astype(jnp.int32).reshape(B // CHUNK, CHUNK)
    return _transe(h2, r2, t2, entity_emb, relation_emb)
